# baseline (speedup 1.0000x reference)
import functools

import jax
import jax.numpy as jnp
from jax import lax
from jax.experimental import pallas as pl
from jax.experimental.pallas import tpu as pltpu
from jax.experimental.pallas import tpu_sc as plsc

D_MODEL = 64
CHUNK = 128
NC, NS = 2, 16
NW = NC * NS
NBUF = 4


@functools.cache
def _make_kernel(n_rows: int):
    assert n_rows % (NW * CHUNK) == 0
    cpw = n_rows // (NW * CHUNK)
    mesh = plsc.VectorSubcoreMesh(core_axis_name="c", subcore_axis_name="s")

    @functools.partial(
        pl.kernel,
        out_type=jax.ShapeDtypeStruct((n_rows, D_MODEL), jnp.float32),
        mesh=mesh,
        scratch_types=[
            pltpu.VMEM((cpw, CHUNK), jnp.int32),
            pltpu.VMEM((NBUF, CHUNK, D_MODEL), jnp.float32),
            pltpu.SemaphoreType.DMA,
            pltpu.SemaphoreType.DMA,
        ],
        compiler_params=pltpu.CompilerParams(use_tc_tiling_on_sc=False),
    )
    def gather_kernel(idx_hbm, table_hbm, out_hbm, idx_v, rows_v, gsem, osem):
        wid = lax.axis_index("s") * NC + lax.axis_index("c")
        base = wid * cpw
        pltpu.sync_copy(idx_hbm.at[pl.ds(base, cpw)], idx_v)

        def gather_start(j, buf):
            pltpu.async_copy(table_hbm.at[idx_v.at[j]], rows_v.at[buf], gsem)

        def gather_wait(j, buf):
            pltpu.make_async_copy(table_hbm.at[idx_v.at[j]], rows_v.at[buf], gsem).wait()

        def out_start(j, buf):
            pltpu.async_copy(
                rows_v.at[buf], out_hbm.at[pl.ds((base + j) * CHUNK, CHUNK)], osem
            )

        def out_wait(buf):
            pltpu.make_async_copy(
                rows_v.at[buf], out_hbm.at[pl.ds(base * CHUNK, CHUNK)], osem
            ).wait()

        gather_start(0, 0)

        def body(j, _):
            buf = j % NBUF
            nxt = (j + 1) % NBUF
            gather_wait(j, buf)

            @pl.when(j + 1 < cpw)
            def _():
                @pl.when(j + 1 >= NBUF)
                def _():
                    out_wait(nxt)

                gather_start(j + 1, nxt)

            out_start(j, buf)
            return 0

        lax.fori_loop(0, cpw, body, 0)
        for k in range(min(NBUF, cpw)):
            out_wait(k)

    return gather_kernel


def kernel(x, table):
    b, s = x.shape
    n_rows = b * s
    idx_grid = x.reshape(n_rows // CHUNK, CHUNK)
    out = _make_kernel(n_rows)(idx_grid, table)
    return out.reshape(b, s, D_MODEL)


# speedup vs baseline: 1.1468x; 1.1468x over previous
import functools

import jax
import jax.numpy as jnp
from jax import lax
from jax.experimental import pallas as pl
from jax.experimental.pallas import tpu as pltpu
from jax.experimental.pallas import tpu_sc as plsc

D_MODEL = 64
DPAD = 128
CHUNK = 128
NC, NS = 2, 16
NW = NC * NS


@functools.cache
def _make_kernel(n_rows: int):
    assert n_rows % (NW * CHUNK) == 0
    n_chunks = n_rows // CHUNK
    cpw = n_chunks // NW
    mesh = plsc.VectorSubcoreMesh(core_axis_name="c", subcore_axis_name="s")

    @functools.partial(
        pl.kernel,
        out_type=jax.ShapeDtypeStruct((n_rows, DPAD), jnp.float32),
        mesh=mesh,
        scratch_types=[
            pltpu.VMEM((cpw, CHUNK), jnp.int32),
            pltpu.VMEM((2, CHUNK, DPAD), jnp.float32),
            pltpu.SemaphoreType.DMA,
        ],
    )
    def gather_kernel(idx_hbm, table_hbm, out_hbm, idx_v, rows_v, gsem):
        wid = lax.axis_index("s") * NC + lax.axis_index("c")
        base = wid * cpw
        pltpu.sync_copy(idx_hbm.at[pl.ds(base, cpw)], idx_v)

        def body(j, _):
            buf = j % 2
            pltpu.async_copy(table_hbm.at[idx_v.at[j]], rows_v.at[buf], gsem).wait()
            pltpu.sync_copy(rows_v.at[buf], out_hbm.at[pl.ds((base + j) * CHUNK, CHUNK)])
            return 0

        lax.fori_loop(0, cpw, body, 0)

    return gather_kernel


def kernel(x, table):
    b, s = x.shape
    n_rows = b * s
    idx = x.astype(jnp.int32).reshape(n_rows // CHUNK, CHUNK)
    tpad = jnp.pad(table, ((0, 0), (0, DPAD - D_MODEL)))
    out = _make_kernel(n_rows)(idx, tpad)
    return out[:, :D_MODEL].reshape(b, s, D_MODEL)


# speedup vs baseline: 1.3132x; 1.1451x over previous
import functools

import jax
import jax.numpy as jnp
from jax import lax
from jax.experimental import pallas as pl
from jax.experimental.pallas import tpu as pltpu
from jax.experimental.pallas import tpu_sc as plsc

D_MODEL = 64
DPAD = 128
CHUNK = 128
NC, NS = 2, 16
NW = NC * NS


@functools.cache
def _make_kernel(n_rows: int):
    assert n_rows % (NW * CHUNK) == 0
    n_chunks = n_rows // CHUNK
    cpw = n_chunks // NW
    mesh = plsc.VectorSubcoreMesh(core_axis_name="c", subcore_axis_name="s")

    NBUF = 4
    LOOK = 2

    @functools.partial(
        pl.kernel,
        out_type=jax.ShapeDtypeStruct((n_rows, DPAD), jnp.float32),
        mesh=mesh,
        scratch_types=[
            pltpu.VMEM((cpw, CHUNK), jnp.int32),
            pltpu.VMEM((NBUF, CHUNK, DPAD), jnp.float32),
            pltpu.SemaphoreType.DMA,
            pltpu.SemaphoreType.DMA,
        ],
    )
    def gather_kernel(idx_hbm, table_hbm, out_hbm, idx_v, rows_v, gsem, osem):
        wid = lax.axis_index("s") * NC + lax.axis_index("c")
        base = wid * cpw
        pltpu.sync_copy(idx_hbm.at[pl.ds(base, cpw)], idx_v)

        def fire(j):
            pltpu.async_copy(table_hbm.at[idx_v.at[j]], rows_v.at[j % NBUF], gsem)

        def put(j):
            pltpu.async_copy(rows_v.at[j % NBUF],
                             out_hbm.at[pl.ds((base + j) * CHUNK, CHUNK)], osem)

        def wait_gather():
            pltpu.make_async_copy(out_hbm.at[pl.ds(0, CHUNK)], rows_v.at[0],
                                  gsem).wait()

        def wait_put():
            pltpu.make_async_copy(rows_v.at[0], out_hbm.at[pl.ds(0, CHUNK)],
                                  osem).wait()

        for j in range(LOOK):
            fire(j)

        def body_warm(j, _):
            fire(j + LOOK)
            wait_gather()
            put(j)
            return 0

        lax.fori_loop(0, NBUF - LOOK, body_warm, 0)

        def body_steady(j, _):
            wait_put()
            fire(j + LOOK)
            wait_gather()
            put(j)
            return 0

        lax.fori_loop(NBUF - LOOK, cpw - LOOK, body_steady, 0)

        def body_tail(j, _):
            wait_gather()
            put(j)
            return 0

        lax.fori_loop(cpw - LOOK, cpw, body_tail, 0)

        for _ in range(NBUF):
            wait_put()

    return gather_kernel


def kernel(x, table):
    b, s = x.shape
    n_rows = b * s
    idx = x.astype(jnp.int32).reshape(n_rows // CHUNK, CHUNK)
    tpad = jnp.pad(table, ((0, 0), (0, DPAD - D_MODEL)))
    out = _make_kernel(n_rows)(idx, tpad)
    return out[:, :D_MODEL].reshape(b, s, D_MODEL)
